# baseline (speedup 1.0000x reference)
import functools

import jax
import jax.numpy as jnp
from jax.experimental import pallas as pl
from jax.experimental.pallas import tpu as pltpu

M, B, DIN, DOUT = 8, 1024, 1024, 1024
BB = 1024
BD = 512


def _mm_kernel(x_ref, w_ref, b_ref, o_ref):
    x = x_ref[0].astype(jnp.bfloat16)
    w = w_ref[0].astype(jnp.bfloat16)
    acc = jax.lax.dot_general(
        x, w, (((1,), (0,)), ((), ())),
        preferred_element_type=jnp.float32,
    )
    o_ref[0] = acc + b_ref[0]


@functools.partial(jax.jit)
def kernel(inputs, W, b):
    grid = (M, DOUT // BD)
    out = pl.pallas_call(
        _mm_kernel,
        grid=grid,
        in_specs=[
            pl.BlockSpec((1, BB, DIN), lambda m, j: (m, 0, 0)),
            pl.BlockSpec((1, DIN, BD), lambda m, j: (m, 0, j)),
            pl.BlockSpec((1, 1, BD), lambda m, j: (m, 0, j)),
        ],
        out_specs=pl.BlockSpec((1, BB, BD), lambda m, j: (m, 0, j)),
        out_shape=jax.ShapeDtypeStruct((M, B, DOUT), jnp.float32),
        compiler_params=pltpu.CompilerParams(
            dimension_semantics=("parallel", "arbitrary"),
        ),
    )(inputs, W, b.reshape(M, 1, DOUT))
    return out


# speedup vs baseline: 1.2547x; 1.2547x over previous
import functools

import jax
import jax.numpy as jnp
from jax.experimental import pallas as pl
from jax.experimental.pallas import tpu as pltpu

M, B, DIN, DOUT = 8, 1024, 1024, 1024


def _mm_kernel(x_ref, w_ref, b_ref, o_ref):
    x = x_ref[0].astype(jnp.bfloat16)
    w = w_ref[0].astype(jnp.bfloat16)
    acc = jax.lax.dot_general(
        x, w, (((1,), (0,)), ((), ())),
        preferred_element_type=jnp.float32,
    )
    o_ref[0] = acc + b_ref[0]


@functools.partial(jax.jit)
def kernel(inputs, W, b):
    grid = (M,)
    out = pl.pallas_call(
        _mm_kernel,
        grid=grid,
        in_specs=[
            pl.BlockSpec((1, B, DIN), lambda m: (m, 0, 0)),
            pl.BlockSpec((1, DIN, DOUT), lambda m: (m, 0, 0)),
            pl.BlockSpec((1, 1, DOUT), lambda m: (m, 0, 0)),
        ],
        out_specs=pl.BlockSpec((1, B, DOUT), lambda m: (m, 0, 0)),
        out_shape=jax.ShapeDtypeStruct((M, B, DOUT), jnp.float32),
        compiler_params=pltpu.CompilerParams(
            dimension_semantics=("arbitrary",),
        ),
    )(inputs, W, b.reshape(M, 1, DOUT))
    return out


# speedup vs baseline: 1.2564x; 1.0014x over previous
import functools

import jax
import jax.numpy as jnp
from jax.experimental import pallas as pl
from jax.experimental.pallas import tpu as pltpu

M, B, DIN, DOUT = 8, 1024, 1024, 1024


def _mm_kernel(x_ref, w_ref, b_ref, o_ref):
    x = x_ref[0]
    w = w_ref[0]
    acc = jax.lax.dot_general(
        x, w, (((1,), (0,)), ((), ())),
        preferred_element_type=jnp.float32,
    )
    o_ref[0] = acc + b_ref[0]


@functools.partial(jax.jit)
def kernel(inputs, W, b):
    grid = (M,)
    out = pl.pallas_call(
        _mm_kernel,
        grid=grid,
        in_specs=[
            pl.BlockSpec((1, B, DIN), lambda m: (m, 0, 0)),
            pl.BlockSpec((1, DIN, DOUT), lambda m: (m, 0, 0)),
            pl.BlockSpec((1, 1, DOUT), lambda m: (m, 0, 0)),
        ],
        out_specs=pl.BlockSpec((1, B, DOUT), lambda m: (m, 0, 0)),
        out_shape=jax.ShapeDtypeStruct((M, B, DOUT), jnp.float32),
        compiler_params=pltpu.CompilerParams(
            dimension_semantics=("arbitrary",),
        ),
    )(inputs, W, b.reshape(M, 1, DOUT))
    return out


# speedup vs baseline: 1.2565x; 1.0001x over previous
import functools

import jax
import jax.numpy as jnp
from jax.experimental import pallas as pl
from jax.experimental.pallas import tpu as pltpu

M, B, DIN, DOUT = 8, 1024, 1024, 1024


def _mm_kernel(x_ref, w_ref, b_ref, o_ref):
    x = x_ref[0]
    w = w_ref[0]
    acc = jax.lax.dot_general(
        x, w, (((1,), (0,)), ((), ())),
        preferred_element_type=jnp.float32,
    )
    o_ref[0] = acc + b_ref[0]


@functools.partial(jax.jit)
def kernel(inputs, W, b):
    grid = (M,)
    out = pl.pallas_call(
        _mm_kernel,
        grid=grid,
        in_specs=[
            pl.BlockSpec((1, B, DIN), lambda m: (m, 0, 0)),
            pl.BlockSpec((1, DIN, DOUT), lambda m: (m, 0, 0)),
            pl.BlockSpec((1, 1, DOUT), lambda m: (m, 0, 0)),
        ],
        out_specs=pl.BlockSpec((1, B, DOUT), lambda m: (m, 0, 0)),
        out_shape=jax.ShapeDtypeStruct((M, B, DOUT), jnp.float32),
        compiler_params=pltpu.CompilerParams(
            dimension_semantics=("parallel",),
        ),
    )(inputs, W, b.reshape(M, 1, DOUT))
    return out
